# baseline (speedup 1.0000x reference)
import functools
import math

import jax
import jax.numpy as jnp
from jax import lax
from jax.experimental import pallas as pl
from jax.experimental.pallas import tpu as pltpu
from jax.experimental.pallas import tpu_sc as plsc

H = 64
HH = H // 2
TD = 32
N = 50000
G = 10
PER = N // G
HALF = PER // 2
EG = 800000
EM = 400000
EU = 2 * EM

EP = 819200
IDXR = EP // 128
PW = EP // 32
PT = EP // 16
CHS = 128
DUMMY = N
ACC_ROWS = 50176
EB = 8192
HB = 8000

_mesh = plsc.VectorSubcoreMesh(core_axis_name="c", subcore_axis_name="s")
_f32 = jnp.float32
_cparams = pltpu.CompilerParams(use_tc_tiling_on_sc=False)


def _sc_gather(table, idx2d):
    chw = 512
    ir = chw // 128
    nch = PW // chw

    @functools.partial(
        pl.kernel, mesh=_mesh, compiler_params=_cparams,
        out_type=jax.ShapeDtypeStruct((EP, H), _f32),
        scratch_types=[pltpu.VMEM((ir, 128), jnp.int32),
                       pltpu.VMEM((ir, 128), jnp.int32),
                       pltpu.VMEM((chw, H), _f32),
                       pltpu.VMEM((chw, H), _f32),
                       pltpu.SemaphoreType.DMA, pltpu.SemaphoreType.DMA,
                       pltpu.SemaphoreType.DMA, pltpu.SemaphoreType.DMA])
    def k(table_hbm, idx_hbm, out_hbm, iva, ivb, ra, rb, ga, gb, sa, sb):
        c = lax.axis_index("c")
        s = lax.axis_index("s")
        w = c * 16 + s
        ib0 = w * (PW // 128)
        rb0 = w * PW

        def load_issue(kk, iv, rows, gsem):
            pltpu.sync_copy(idx_hbm.at[pl.ds(ib0 + kk * ir, ir)], iv)
            for j in range(ir):
                pltpu.async_copy(table_hbm.at[iv.at[j]],
                                 rows.at[pl.ds(j * 128, 128)], gsem)

        load_issue(0, iva, ra, ga)
        load_issue(1, ivb, rb, gb)

        def body(g, carry):
            for kk, iv, rows, gsem, ssem in ((2 * g, iva, ra, ga, sa),
                                             (2 * g + 1, ivb, rb, gb, sb)):
                for j in range(ir):
                    pltpu.make_async_copy(
                        table_hbm.at[iv.at[j]],
                        rows.at[pl.ds(j * 128, 128)], gsem).wait()
                out_slab = out_hbm.at[pl.ds(rb0 + kk * chw, chw)]
                pltpu.async_copy(rows, out_slab, ssem)

                @pl.when(kk + 2 < nch)
                def _():
                    pltpu.make_async_copy(rows, out_slab, ssem).wait()
                    load_issue(kk + 2, iv, rows, gsem)
            return carry

        lax.fori_loop(0, nch // 2, body, 0)
        pltpu.make_async_copy(
            ra, out_hbm.at[pl.ds(rb0 + (nch - 2) * chw, chw)], sa).wait()
        pltpu.make_async_copy(
            rb, out_hbm.at[pl.ds(rb0 + (nch - 1) * chw, chw)], sb).wait()

    return k(table, idx2d)


def _acc_prologue(s, accum, zeros_hbm):
    for m in range(7):
        kz = s + 16 * m

        @pl.when(kz < ACC_ROWS // 512)
        def _():
            pltpu.sync_copy(zeros_hbm, accum.at[pl.ds(kz * 512, 512)])

    plsc.subcore_barrier()


def _acc_epilogue(s, accum, out_hbm):
    plsc.subcore_barrier()
    for m in range(4):
        kz = s + 16 * m

        @pl.when(kz < N // 1000)
        def _():
            pltpu.sync_copy(accum.at[pl.ds(kz * 1000, 1000)],
                            out_hbm.at[pl.ds(kz * 1000, 1000)])


def _sc_scatter_add(msgl, msgr, dstl, zeros):
    @functools.partial(
        pl.kernel, mesh=_mesh, compiler_params=_cparams,
        out_type=[jax.ShapeDtypeStruct((N, HH), _f32)] * 2,
        scratch_types=[pltpu.VMEM((1, 128), jnp.int32),
                       pltpu.VMEM((1, 128), jnp.int32),
                       pltpu.VMEM((CHS, HH), _f32),
                       pltpu.VMEM((CHS, HH), _f32),
                       pltpu.VMEM_SHARED((ACC_ROWS, HH), _f32),
                       pltpu.SemaphoreType.DMA, pltpu.SemaphoreType.DMA])
    def k(msgl_hbm, msgr_hbm, idx_hbm, zeros_hbm, outl_hbm, outr_hbm,
          iva, ivb, ra, rb, accum, la, lb):
        c = lax.axis_index("c")
        s = lax.axis_index("s")
        ib0 = s * (PT // 128)
        rb0 = s * PT
        nch = PT // CHS
        _acc_prologue(s, accum, zeros_hbm)

        def run(msg_hbm, out_hbm):
            def issue(kk, rows, lsem):
                pltpu.async_copy(msg_hbm.at[pl.ds(rb0 + kk * CHS, CHS)],
                                 rows, lsem)

            issue(0, ra, la)
            issue(1, rb, lb)

            def body(g, carry):
                for kk, iv, rows, lsem in ((2 * g, iva, ra, la),
                                           (2 * g + 1, ivb, rb, lb)):
                    pltpu.make_async_copy(
                        msg_hbm.at[pl.ds(rb0 + kk * CHS, CHS)], rows,
                        lsem).wait()
                    pltpu.sync_copy(idx_hbm.at[pl.ds(ib0 + kk, 1)], iv)
                    pltpu.sync_copy(rows, accum.at[iv.at[0]], add=True)

                    @pl.when(kk + 2 < nch)
                    def _():
                        issue(kk + 2, rows, lsem)
                return carry

            lax.fori_loop(0, nch // 2, body, 0)
            _acc_epilogue(s, accum, out_hbm)

        @pl.when(c == 0)
        def _():
            run(msgl_hbm, outl_hbm)

        @pl.when(c == 1)
        def _():
            run(msgr_hbm, outr_hbm)

    return k(msgl, msgr, dstl, zeros)


def _sc_gin_agg(xl, xr, src2d, dstl, zeros):
    @functools.partial(
        pl.kernel, mesh=_mesh, compiler_params=_cparams,
        out_type=[jax.ShapeDtypeStruct((N, HH), _f32)] * 2,
        scratch_types=[pltpu.VMEM((1, 128), jnp.int32),
                       pltpu.VMEM((1, 128), jnp.int32),
                       pltpu.VMEM((1, 128), jnp.int32),
                       pltpu.VMEM((1, 128), jnp.int32),
                       pltpu.VMEM((CHS, HH), _f32),
                       pltpu.VMEM((CHS, HH), _f32),
                       pltpu.VMEM_SHARED((ACC_ROWS, HH), _f32),
                       pltpu.SemaphoreType.DMA, pltpu.SemaphoreType.DMA])
    def k(xl_hbm, xr_hbm, src_hbm, idx_hbm, zeros_hbm, outl_hbm, outr_hbm,
          isa, isb, iva, ivb, ra, rb, accum, ga, gb):
        c = lax.axis_index("c")
        s = lax.axis_index("s")
        ib0 = s * (PT // 128)
        nch = PT // CHS
        _acc_prologue(s, accum, zeros_hbm)

        def run(x_hbm, out_hbm):
            def issue(kk, isv, rows, gsem):
                pltpu.sync_copy(src_hbm.at[pl.ds(ib0 + kk, 1)], isv)
                pltpu.async_copy(x_hbm.at[isv.at[0]], rows, gsem)

            issue(0, isa, ra, ga)
            issue(1, isb, rb, gb)

            def body(g, carry):
                for kk, isv, iv, rows, gsem in ((2 * g, isa, iva, ra, ga),
                                                (2 * g + 1, isb, ivb, rb, gb)):
                    pltpu.make_async_copy(x_hbm.at[isv.at[0]], rows,
                                          gsem).wait()
                    pltpu.sync_copy(idx_hbm.at[pl.ds(ib0 + kk, 1)], iv)
                    pltpu.sync_copy(rows, accum.at[iv.at[0]], add=True)

                    @pl.when(kk + 2 < nch)
                    def _():
                        issue(kk + 2, isv, rows, gsem)
                return carry

            lax.fori_loop(0, nch // 2, body, 0)
            _acc_epilogue(s, accum, out_hbm)

        @pl.when(c == 0)
        def _():
            run(xl_hbm, outl_hbm)

        @pl.when(c == 1)
        def _():
            run(xr_hbm, outr_hbm)

    return k(xl, xr, src2d, dstl, zeros)


def _full(shape):
    return pl.BlockSpec(shape, lambda *_: tuple(0 for _ in shape))


def _time_body(t_ref, w1, b1, w2, b2, T0, bT0, T1, bT1, te0, te1):
    tcol = t_ref[...]
    j = lax.broadcasted_iota(jnp.int32, (1, TD), 1).astype(_f32)
    freqs = jnp.exp(-math.log(10000.0) * j / float(TD))
    a = tcol * freqs
    emb = jnp.concatenate([jnp.cos(a), jnp.sin(a)], axis=1)
    h = jnp.maximum(emb @ w1[...] + b1[...], 0.0)
    tm = h @ w2[...] + b2[...]
    te0[...] = tm @ T0[...] + bT0[...]
    te1[...] = tm @ T1[...] + bT1[...]


def _time_emb(tpad, tp, l0, l1):
    out = pl.pallas_call(
        _time_body,
        in_specs=[_full((16, 1)), _full((H, TD)), _full((1, TD)),
                  _full((TD, TD)), _full((1, TD)),
                  _full((TD, H)), _full((1, H)), _full((TD, H)), _full((1, H))],
        out_specs=[_full((16, H)), _full((16, H))],
        out_shape=[jax.ShapeDtypeStruct((16, H), _f32)] * 2,
    )
    return out(tpad, tp["W1"], tp["b1"][None, :], tp["W2"], tp["b2"][None, :],
               l0["T"], l0["bT"][None, :], l1["T"], l1["bT"][None, :])


def _embed_body(au_ref, W, b, o_ref):
    ji = lax.broadcasted_iota(jnp.int32, (1, H), 1)
    jf = jnp.floor(ji.astype(_f32) / 2.0)
    inv_dim_t = jnp.exp(-(math.log(10000.0) * 2.0 / float(H)) * jf)
    off = jnp.where(ji % 2 == 1, math.pi / 2.0, 0.0).astype(_f32)
    pos = au_ref[...] * inv_dim_t
    o_ref[...] = jnp.sin(pos + off) @ W[...] + b[...]


def _edge_embed(au, ep):
    out = pl.pallas_call(
        _embed_body,
        grid=(EP // EB,),
        in_specs=[pl.BlockSpec((EB, 1), lambda i: (i, 0)),
                  _full((H, H)), _full((1, H))],
        out_specs=pl.BlockSpec((EB, H), lambda i: (i, 0)),
        out_shape=jax.ShapeDtypeStruct((EP, H), _f32),
    )
    return out(au, ep["W"], ep["b"][None, :])


def _node_common(x, al, ar, W1, b1, W2, b2, eps, gw, gb, gms):
    agg = jnp.concatenate([al[...][0], ar[...][0]], axis=1)
    h = (1.0 + eps[0, 0]) * x[...][0] + agg
    h = jnp.maximum(h @ W1[...] + b1[...], 0.0)
    h = h @ W2[...] + b2[...]
    mean = jnp.mean(h, axis=0, keepdims=True)
    sub = h - mean * gms[...]
    var = jnp.mean(sub * sub, axis=0, keepdims=True)
    return jnp.maximum(gw[...] * sub * jax.lax.rsqrt(var + 1e-5) + gb[...],
                       0.0)


def _node0_body(x, al, ar, te, W1, b1, W2, b2, eps, gw, gb, gms,
                A, bA, B, bB, V, bV, U, bU,
                f_o, xa_o, xb_o, xv_o, xu_o):
    f = _node_common(x, al, ar, W1, b1, W2, b2, eps, gw, gb, gms)
    f_o[...] = f[None]
    xa_o[...] = (f @ A[...] + bA[...] + te[...][0])[None]
    xb_o[...] = (f @ B[...] + bB[...])[None]
    xv_o[...] = (f @ V[...] + bV[...])[None]
    xu_o[...] = (f @ U[...] + bU[...])[None]


def _node1_body(x, al, ar, te, W1, b1, W2, b2, eps, gw, gb, gms,
                A, bA, B, bB, xa_o, xb_o):
    f = _node_common(x, al, ar, W1, b1, W2, b2, eps, gw, gb, gms)
    xa_o[...] = (f @ A[...] + bA[...] + te[...][0])[None]
    xb_o[...] = (f @ B[...] + bB[...])[None]


def _node_dense(x, aggl, aggr, te, lp, first):
    g = lp["gin"]
    n = lp["gn"]
    a = lp["agnn"]
    blk = pl.BlockSpec((1, HALF, H), lambda i: (i, 0, 0))
    blkh = pl.BlockSpec((1, HALF, HH), lambda i: (i, 0, 0))
    common_specs = [blk, blkh, blkh,
                    pl.BlockSpec((1, 1, H), lambda i: (i // 2, 0, 0)),
                    _full((H, H)), _full((1, H)), _full((H, H)), _full((1, H)),
                    _full((1, 1)), _full((1, H)), _full((1, H)), _full((1, H)),
                    _full((H, H)), _full((1, H)), _full((H, H)), _full((1, H))]
    common_args = (x.reshape(N // HALF, HALF, H),
                   aggl.reshape(N // HALF, HALF, HH),
                   aggr.reshape(N // HALF, HALF, HH), te.reshape(16, 1, H),
                   g["W1"], g["b1"][None, :], g["W2"], g["b2"][None, :],
                   g["eps"].reshape(1, 1),
                   n["weight"][None, :], n["bias"][None, :],
                   n["mean_scale"][None, :],
                   a["A"], a["bA"][None, :], a["B"], a["bB"][None, :])
    if first:
        out = pl.pallas_call(
            _node0_body,
            grid=(N // HALF,),
            in_specs=common_specs + [_full((H, H)), _full((1, H)),
                                     _full((H, H)), _full((1, H))],
            out_specs=[blk] * 5,
            out_shape=[jax.ShapeDtypeStruct((N // HALF, HALF, H), _f32)] * 5,
        )
        return out(*common_args, a["V"], a["bV"][None, :],
                   a["U"], a["bU"][None, :])
    out = pl.pallas_call(
        _node1_body,
        grid=(N // HALF,),
        in_specs=common_specs,
        out_specs=[blk, blk],
        out_shape=[jax.ShapeDtypeStruct((N // HALF, HALF, H), _f32)] * 2,
    )
    return out(*common_args)


def _edge0_body(e, ga, gb, gv, C, bC, e1_o, ml_o, mr_o):
    en = ga[...] + gb[...] + e[...] @ C[...] + bC[...]
    gate = jax.nn.sigmoid(en)
    msg = gate * gv[...]
    ml_o[...] = msg[:, :HH]
    mr_o[...] = msg[:, HH:]
    e1_o[...] = e[...] + jnp.maximum(en, 0.0)


def _edge_dense0(e, ga, gb, gv, a):
    blk = pl.BlockSpec((EB, H), lambda i: (i, 0))
    blkh = pl.BlockSpec((EB, HH), lambda i: (i, 0))
    out = pl.pallas_call(
        _edge0_body,
        grid=(EP // EB,),
        in_specs=[blk, blk, blk, blk, _full((H, H)), _full((1, H))],
        out_specs=[blk, blkh, blkh],
        out_shape=[jax.ShapeDtypeStruct((EP, H), _f32),
                   jax.ShapeDtypeStruct((EP, HH), _f32),
                   jax.ShapeDtypeStruct((EP, HH), _f32)],
    )
    return out(e, ga, gb, gv, a["C"], a["bC"][None, :])


def _edge1_body(e, ga, gb, C, bC, e1_o):
    en = ga[...] + gb[...] + e[...] @ C[...] + bC[...]
    e1_o[...] = e[...] + jnp.maximum(en, 0.0)


def _edge_dense1(e, ga, gb, a):
    blk = pl.BlockSpec((EB, H), lambda i: (i, 0))
    out = pl.pallas_call(
        _edge1_body,
        grid=(EP // EB,),
        in_specs=[blk, blk, blk, _full((H, H)), _full((1, H))],
        out_specs=blk,
        out_shape=jax.ShapeDtypeStruct((EP, H), _f32),
    )
    return out(e, ga, gb, a["C"], a["bC"][None, :])


def _xupd_body(f, xu, al, ar, o):
    agg2 = jnp.concatenate([al[...][0], ar[...][0]], axis=1)
    o[...] = f[...] + (jnp.maximum(xu[...][0] + agg2, 0.0))[None]


def _xupd(f, xu, aggl, aggr):
    blk = pl.BlockSpec((1, HALF, H), lambda i: (i, 0, 0))
    blkh = pl.BlockSpec((1, HALF, HH), lambda i: (i, 0, 0))
    out = pl.pallas_call(
        _xupd_body,
        grid=(N // HALF,),
        in_specs=[blk, blk, blkh, blkh],
        out_specs=blk,
        out_shape=jax.ShapeDtypeStruct((N // HALF, HALF, H), _f32),
    )
    return out(f, xu, aggl.reshape(N // HALF, HALF, HH),
               aggr.reshape(N // HALF, HALF, HH)).reshape(N, H)


def _head_body(ea, eb, W1, b1, W2, b2, W3, b3, o):
    def mlp(z):
        z = jnp.maximum(z @ W1[...] + b1[...], 0.0)
        z = jnp.maximum(z @ W2[...] + b2[...], 0.0)
        return z @ W3[...] + b3[...]

    o[...] = 0.5 * (mlp(ea[...]) + mlp(eb[...]))


def _head(e2, mp):
    out = pl.pallas_call(
        _head_body,
        grid=(EM // HB,),
        in_specs=[pl.BlockSpec((HB, H), lambda i: (i, 0)),
                  pl.BlockSpec((HB, H), lambda i: (i + EM // HB, 0)),
                  _full((H, 2 * H)), _full((1, 2 * H)),
                  _full((2 * H, H)), _full((1, H)),
                  _full((H, 1)), _full((1, 1))],
        out_specs=pl.BlockSpec((HB, 1), lambda i: (i, 0)),
        out_shape=jax.ShapeDtypeStruct((EM, 1), _f32),
    )
    return out(e2, e2, mp["W1"], mp["b1"][None, :], mp["W2"], mp["b2"][None, :],
               mp["W3"], mp["b3"].reshape(1, 1))


def kernel(x, edge_index, batch, x_indicator, edge_index_mapping,
           noise_mapping_attr, t, params):
    del batch, x_indicator
    padE = EP - EU
    zpad = jnp.zeros((padE,), jnp.int32)
    npad = jnp.full((padE,), N, jnp.int32)

    src_g = jnp.concatenate([edge_index[0], zpad]).reshape(IDXR, 128)
    dst_g = jnp.concatenate([edge_index[1], npad]).reshape(IDXR, 128)
    src_u = jnp.concatenate(
        [edge_index_mapping[0], edge_index_mapping[1], zpad]).reshape(IDXR, 128)
    dst_u_raw = jnp.concatenate([edge_index_mapping[1], edge_index_mapping[0]])
    dst_u = jnp.concatenate([dst_u_raw, zpad]).reshape(IDXR, 128)
    dst_u_dummy = jnp.concatenate([dst_u_raw, npad]).reshape(IDXR, 128)

    au = jnp.concatenate([noise_mapping_attr, noise_mapping_attr,
                          jnp.zeros((padE,), _f32)]).reshape(EP, 1)
    tpad = jnp.pad(t, (0, 16 - G)).reshape(16, 1)
    zeros = jnp.zeros((512, HH), _f32)

    lp0, lp1 = params["layers"]
    te0, te1 = _time_emb(tpad, params["time"], lp0["agnn"], lp1["agnn"])

    e = _edge_embed(au, params["edge_embed"])

    aggl, aggr = _sc_gin_agg(x[:, :HH], x[:, HH:], src_g, dst_g, zeros)
    f0, xa0, xb0, xv0, xu0 = _node_dense(x, aggl, aggr, te0, lp0, first=True)
    ga = _sc_gather(xa0.reshape(N, H), src_u)
    gb = _sc_gather(xb0.reshape(N, H), dst_u)
    gv = _sc_gather(xv0.reshape(N, H), src_u)
    e, msgl, msgr = _edge_dense0(e, ga, gb, gv, lp0["agnn"])
    a2l, a2r = _sc_scatter_add(msgl, msgr, dst_u_dummy, zeros)
    x1 = _xupd(f0, xu0, a2l, a2r)

    bgl, bgr = _sc_gin_agg(x1[:, :HH], x1[:, HH:], src_g, dst_g, zeros)
    xa1, xb1 = _node_dense(x1, bgl, bgr, te1, lp1, first=False)
    ga1 = _sc_gather(xa1.reshape(N, H), src_u)
    gb1 = _sc_gather(xb1.reshape(N, H), dst_u)
    e = _edge_dense1(e, ga1, gb1, lp1["agnn"])

    return _head(e, params["map"])


# speedup vs baseline: 1.0014x; 1.0014x over previous
import functools
import math

import jax
import jax.numpy as jnp
from jax import lax
from jax.experimental import pallas as pl
from jax.experimental.pallas import tpu as pltpu
from jax.experimental.pallas import tpu_sc as plsc

H = 64
HH = H // 2
TD = 32
N = 50000
G = 10
PER = N // G
HALF = PER // 2
EG = 800000
EM = 400000
EU = 2 * EM

EP = 819200
IDXR = EP // 128
PW = EP // 32
PT = EP // 16
CHS = 128
DUMMY = N
ACC_ROWS = 50176
EB = 8192
HB = 8000

_mesh = plsc.VectorSubcoreMesh(core_axis_name="c", subcore_axis_name="s")
_f32 = jnp.float32
_cparams = pltpu.CompilerParams(use_tc_tiling_on_sc=False)


_NBUF = 4


def _sc_gather(table, idx2d):
    chw = 256
    ir = chw // 128
    nch = PW // chw

    @functools.partial(
        pl.kernel, mesh=_mesh, compiler_params=_cparams,
        out_type=jax.ShapeDtypeStruct((EP, H), _f32),
        scratch_types=[[pltpu.VMEM((ir, 128), jnp.int32)] * _NBUF,
                       [pltpu.VMEM((chw, H), _f32)] * _NBUF,
                       [pltpu.SemaphoreType.DMA] * _NBUF,
                       [pltpu.SemaphoreType.DMA] * _NBUF])
    def k(table_hbm, idx_hbm, out_hbm, ivs, rbs, gsems, ssems):
        c = lax.axis_index("c")
        s = lax.axis_index("s")
        w = c * 16 + s
        ib0 = w * (PW // 128)
        rb0 = w * PW

        def load_issue(kk, iv, rows, gsem):
            pltpu.sync_copy(idx_hbm.at[pl.ds(ib0 + kk * ir, ir)], iv)
            for j in range(ir):
                pltpu.async_copy(table_hbm.at[iv.at[j]],
                                 rows.at[pl.ds(j * 128, 128)], gsem)

        for o in range(_NBUF):
            load_issue(o, ivs[o], rbs[o], gsems[o])

        def body(g, carry):
            for o in range(_NBUF):
                kk = _NBUF * g + o
                iv, rows, gsem, ssem = ivs[o], rbs[o], gsems[o], ssems[o]
                for j in range(ir):
                    pltpu.make_async_copy(
                        table_hbm.at[iv.at[j]],
                        rows.at[pl.ds(j * 128, 128)], gsem).wait()
                out_slab = out_hbm.at[pl.ds(rb0 + kk * chw, chw)]
                pltpu.async_copy(rows, out_slab, ssem)

                @pl.when(kk + _NBUF < nch)
                def _():
                    pltpu.make_async_copy(rows, out_slab, ssem).wait()
                    load_issue(kk + _NBUF, iv, rows, gsem)
            return carry

        lax.fori_loop(0, nch // _NBUF, body, 0)
        for o in range(_NBUF):
            kk = nch - _NBUF + o
            pltpu.make_async_copy(
                rbs[o], out_hbm.at[pl.ds(rb0 + kk * chw, chw)], ssems[o]).wait()

    return k(table, idx2d)


def _acc_prologue(s, accum, zeros_hbm):
    for m in range(7):
        kz = s + 16 * m

        @pl.when(kz < ACC_ROWS // 512)
        def _():
            pltpu.sync_copy(zeros_hbm, accum.at[pl.ds(kz * 512, 512)])

    plsc.subcore_barrier()


def _acc_epilogue(s, accum, out_hbm):
    plsc.subcore_barrier()
    for m in range(4):
        kz = s + 16 * m

        @pl.when(kz < N // 1000)
        def _():
            pltpu.sync_copy(accum.at[pl.ds(kz * 1000, 1000)],
                            out_hbm.at[pl.ds(kz * 1000, 1000)])


def _sc_scatter_add(msgl, msgr, dstl, zeros):
    @functools.partial(
        pl.kernel, mesh=_mesh, compiler_params=_cparams,
        out_type=[jax.ShapeDtypeStruct((N, HH), _f32)] * 2,
        scratch_types=[pltpu.VMEM((1, 128), jnp.int32),
                       pltpu.VMEM((1, 128), jnp.int32),
                       pltpu.VMEM((CHS, HH), _f32),
                       pltpu.VMEM((CHS, HH), _f32),
                       pltpu.VMEM_SHARED((ACC_ROWS, HH), _f32),
                       pltpu.SemaphoreType.DMA, pltpu.SemaphoreType.DMA])
    def k(msgl_hbm, msgr_hbm, idx_hbm, zeros_hbm, outl_hbm, outr_hbm,
          iva, ivb, ra, rb, accum, la, lb):
        c = lax.axis_index("c")
        s = lax.axis_index("s")
        ib0 = s * (PT // 128)
        rb0 = s * PT
        nch = PT // CHS
        _acc_prologue(s, accum, zeros_hbm)

        def run(msg_hbm, out_hbm):
            def issue(kk, rows, lsem):
                pltpu.async_copy(msg_hbm.at[pl.ds(rb0 + kk * CHS, CHS)],
                                 rows, lsem)

            issue(0, ra, la)
            issue(1, rb, lb)

            def body(g, carry):
                for kk, iv, rows, lsem in ((2 * g, iva, ra, la),
                                           (2 * g + 1, ivb, rb, lb)):
                    pltpu.make_async_copy(
                        msg_hbm.at[pl.ds(rb0 + kk * CHS, CHS)], rows,
                        lsem).wait()
                    pltpu.sync_copy(idx_hbm.at[pl.ds(ib0 + kk, 1)], iv)
                    pltpu.sync_copy(rows, accum.at[iv.at[0]], add=True)

                    @pl.when(kk + 2 < nch)
                    def _():
                        issue(kk + 2, rows, lsem)
                return carry

            lax.fori_loop(0, nch // 2, body, 0)
            _acc_epilogue(s, accum, out_hbm)

        @pl.when(c == 0)
        def _():
            run(msgl_hbm, outl_hbm)

        @pl.when(c == 1)
        def _():
            run(msgr_hbm, outr_hbm)

    return k(msgl, msgr, dstl, zeros)


def _sc_gin_agg(xl, xr, src2d, dstl, zeros):
    @functools.partial(
        pl.kernel, mesh=_mesh, compiler_params=_cparams,
        out_type=[jax.ShapeDtypeStruct((N, HH), _f32)] * 2,
        scratch_types=[pltpu.VMEM((1, 128), jnp.int32),
                       pltpu.VMEM((1, 128), jnp.int32),
                       pltpu.VMEM((1, 128), jnp.int32),
                       pltpu.VMEM((1, 128), jnp.int32),
                       pltpu.VMEM((CHS, HH), _f32),
                       pltpu.VMEM((CHS, HH), _f32),
                       pltpu.VMEM_SHARED((ACC_ROWS, HH), _f32),
                       pltpu.SemaphoreType.DMA, pltpu.SemaphoreType.DMA])
    def k(xl_hbm, xr_hbm, src_hbm, idx_hbm, zeros_hbm, outl_hbm, outr_hbm,
          isa, isb, iva, ivb, ra, rb, accum, ga, gb):
        c = lax.axis_index("c")
        s = lax.axis_index("s")
        ib0 = s * (PT // 128)
        nch = PT // CHS
        _acc_prologue(s, accum, zeros_hbm)

        def run(x_hbm, out_hbm):
            def issue(kk, isv, rows, gsem):
                pltpu.sync_copy(src_hbm.at[pl.ds(ib0 + kk, 1)], isv)
                pltpu.async_copy(x_hbm.at[isv.at[0]], rows, gsem)

            issue(0, isa, ra, ga)
            issue(1, isb, rb, gb)

            def body(g, carry):
                for kk, isv, iv, rows, gsem in ((2 * g, isa, iva, ra, ga),
                                                (2 * g + 1, isb, ivb, rb, gb)):
                    pltpu.make_async_copy(x_hbm.at[isv.at[0]], rows,
                                          gsem).wait()
                    pltpu.sync_copy(idx_hbm.at[pl.ds(ib0 + kk, 1)], iv)
                    pltpu.sync_copy(rows, accum.at[iv.at[0]], add=True)

                    @pl.when(kk + 2 < nch)
                    def _():
                        issue(kk + 2, isv, rows, gsem)
                return carry

            lax.fori_loop(0, nch // 2, body, 0)
            _acc_epilogue(s, accum, out_hbm)

        @pl.when(c == 0)
        def _():
            run(xl_hbm, outl_hbm)

        @pl.when(c == 1)
        def _():
            run(xr_hbm, outr_hbm)

    return k(xl, xr, src2d, dstl, zeros)


def _full(shape):
    return pl.BlockSpec(shape, lambda *_: tuple(0 for _ in shape))


def _time_body(t_ref, w1, b1, w2, b2, T0, bT0, T1, bT1, te0, te1):
    tcol = t_ref[...]
    j = lax.broadcasted_iota(jnp.int32, (1, TD), 1).astype(_f32)
    freqs = jnp.exp(-math.log(10000.0) * j / float(TD))
    a = tcol * freqs
    emb = jnp.concatenate([jnp.cos(a), jnp.sin(a)], axis=1)
    h = jnp.maximum(emb @ w1[...] + b1[...], 0.0)
    tm = h @ w2[...] + b2[...]
    te0[...] = tm @ T0[...] + bT0[...]
    te1[...] = tm @ T1[...] + bT1[...]


def _time_emb(tpad, tp, l0, l1):
    out = pl.pallas_call(
        _time_body,
        in_specs=[_full((16, 1)), _full((H, TD)), _full((1, TD)),
                  _full((TD, TD)), _full((1, TD)),
                  _full((TD, H)), _full((1, H)), _full((TD, H)), _full((1, H))],
        out_specs=[_full((16, H)), _full((16, H))],
        out_shape=[jax.ShapeDtypeStruct((16, H), _f32)] * 2,
    )
    return out(tpad, tp["W1"], tp["b1"][None, :], tp["W2"], tp["b2"][None, :],
               l0["T"], l0["bT"][None, :], l1["T"], l1["bT"][None, :])


def _embed_body(au_ref, W, b, o_ref):
    ji = lax.broadcasted_iota(jnp.int32, (1, H), 1)
    jf = jnp.floor(ji.astype(_f32) / 2.0)
    inv_dim_t = jnp.exp(-(math.log(10000.0) * 2.0 / float(H)) * jf)
    off = jnp.where(ji % 2 == 1, math.pi / 2.0, 0.0).astype(_f32)
    pos = au_ref[...] * inv_dim_t
    o_ref[...] = jnp.sin(pos + off) @ W[...] + b[...]


def _edge_embed(au, ep):
    out = pl.pallas_call(
        _embed_body,
        grid=(EP // EB,),
        in_specs=[pl.BlockSpec((EB, 1), lambda i: (i, 0)),
                  _full((H, H)), _full((1, H))],
        out_specs=pl.BlockSpec((EB, H), lambda i: (i, 0)),
        out_shape=jax.ShapeDtypeStruct((EP, H), _f32),
    )
    return out(au, ep["W"], ep["b"][None, :])


def _node_common(x, al, ar, W1, b1, W2, b2, eps, gw, gb, gms):
    agg = jnp.concatenate([al[...][0], ar[...][0]], axis=1)
    h = (1.0 + eps[0, 0]) * x[...][0] + agg
    h = jnp.maximum(h @ W1[...] + b1[...], 0.0)
    h = h @ W2[...] + b2[...]
    mean = jnp.mean(h, axis=0, keepdims=True)
    sub = h - mean * gms[...]
    var = jnp.mean(sub * sub, axis=0, keepdims=True)
    return jnp.maximum(gw[...] * sub * jax.lax.rsqrt(var + 1e-5) + gb[...],
                       0.0)


def _node0_body(x, al, ar, te, W1, b1, W2, b2, eps, gw, gb, gms,
                A, bA, B, bB, V, bV, U, bU,
                f_o, xa_o, xb_o, xv_o, xu_o):
    f = _node_common(x, al, ar, W1, b1, W2, b2, eps, gw, gb, gms)
    f_o[...] = f[None]
    xa_o[...] = (f @ A[...] + bA[...] + te[...][0])[None]
    xb_o[...] = (f @ B[...] + bB[...])[None]
    xv_o[...] = (f @ V[...] + bV[...])[None]
    xu_o[...] = (f @ U[...] + bU[...])[None]


def _node1_body(x, al, ar, te, W1, b1, W2, b2, eps, gw, gb, gms,
                A, bA, B, bB, xa_o, xb_o):
    f = _node_common(x, al, ar, W1, b1, W2, b2, eps, gw, gb, gms)
    xa_o[...] = (f @ A[...] + bA[...] + te[...][0])[None]
    xb_o[...] = (f @ B[...] + bB[...])[None]


def _node_dense(x, aggl, aggr, te, lp, first):
    g = lp["gin"]
    n = lp["gn"]
    a = lp["agnn"]
    blk = pl.BlockSpec((1, HALF, H), lambda i: (i, 0, 0))
    blkh = pl.BlockSpec((1, HALF, HH), lambda i: (i, 0, 0))
    common_specs = [blk, blkh, blkh,
                    pl.BlockSpec((1, 1, H), lambda i: (i // 2, 0, 0)),
                    _full((H, H)), _full((1, H)), _full((H, H)), _full((1, H)),
                    _full((1, 1)), _full((1, H)), _full((1, H)), _full((1, H)),
                    _full((H, H)), _full((1, H)), _full((H, H)), _full((1, H))]
    common_args = (x.reshape(N // HALF, HALF, H),
                   aggl.reshape(N // HALF, HALF, HH),
                   aggr.reshape(N // HALF, HALF, HH), te.reshape(16, 1, H),
                   g["W1"], g["b1"][None, :], g["W2"], g["b2"][None, :],
                   g["eps"].reshape(1, 1),
                   n["weight"][None, :], n["bias"][None, :],
                   n["mean_scale"][None, :],
                   a["A"], a["bA"][None, :], a["B"], a["bB"][None, :])
    if first:
        out = pl.pallas_call(
            _node0_body,
            grid=(N // HALF,),
            in_specs=common_specs + [_full((H, H)), _full((1, H)),
                                     _full((H, H)), _full((1, H))],
            out_specs=[blk] * 5,
            out_shape=[jax.ShapeDtypeStruct((N // HALF, HALF, H), _f32)] * 5,
        )
        return out(*common_args, a["V"], a["bV"][None, :],
                   a["U"], a["bU"][None, :])
    out = pl.pallas_call(
        _node1_body,
        grid=(N // HALF,),
        in_specs=common_specs,
        out_specs=[blk, blk],
        out_shape=[jax.ShapeDtypeStruct((N // HALF, HALF, H), _f32)] * 2,
    )
    return out(*common_args)


def _edge0_body(e, ga, gb, gv, C, bC, e1_o, ml_o, mr_o):
    en = ga[...] + gb[...] + e[...] @ C[...] + bC[...]
    gate = jax.nn.sigmoid(en)
    msg = gate * gv[...]
    ml_o[...] = msg[:, :HH]
    mr_o[...] = msg[:, HH:]
    e1_o[...] = e[...] + jnp.maximum(en, 0.0)


def _edge_dense0(e, ga, gb, gv, a):
    blk = pl.BlockSpec((EB, H), lambda i: (i, 0))
    blkh = pl.BlockSpec((EB, HH), lambda i: (i, 0))
    out = pl.pallas_call(
        _edge0_body,
        grid=(EP // EB,),
        in_specs=[blk, blk, blk, blk, _full((H, H)), _full((1, H))],
        out_specs=[blk, blkh, blkh],
        out_shape=[jax.ShapeDtypeStruct((EP, H), _f32),
                   jax.ShapeDtypeStruct((EP, HH), _f32),
                   jax.ShapeDtypeStruct((EP, HH), _f32)],
    )
    return out(e, ga, gb, gv, a["C"], a["bC"][None, :])


def _edge1_body(e, ga, gb, C, bC, e1_o):
    en = ga[...] + gb[...] + e[...] @ C[...] + bC[...]
    e1_o[...] = e[...] + jnp.maximum(en, 0.0)


def _edge_dense1(e, ga, gb, a):
    blk = pl.BlockSpec((EB, H), lambda i: (i, 0))
    out = pl.pallas_call(
        _edge1_body,
        grid=(EP // EB,),
        in_specs=[blk, blk, blk, _full((H, H)), _full((1, H))],
        out_specs=blk,
        out_shape=jax.ShapeDtypeStruct((EP, H), _f32),
    )
    return out(e, ga, gb, a["C"], a["bC"][None, :])


def _xupd_body(f, xu, al, ar, o):
    agg2 = jnp.concatenate([al[...][0], ar[...][0]], axis=1)
    o[...] = f[...] + (jnp.maximum(xu[...][0] + agg2, 0.0))[None]


def _xupd(f, xu, aggl, aggr):
    blk = pl.BlockSpec((1, HALF, H), lambda i: (i, 0, 0))
    blkh = pl.BlockSpec((1, HALF, HH), lambda i: (i, 0, 0))
    out = pl.pallas_call(
        _xupd_body,
        grid=(N // HALF,),
        in_specs=[blk, blk, blkh, blkh],
        out_specs=blk,
        out_shape=jax.ShapeDtypeStruct((N // HALF, HALF, H), _f32),
    )
    return out(f, xu, aggl.reshape(N // HALF, HALF, HH),
               aggr.reshape(N // HALF, HALF, HH)).reshape(N, H)


def _head_body(ea, eb, W1, b1, W2, b2, W3, b3, o):
    def mlp(z):
        z = jnp.maximum(z @ W1[...] + b1[...], 0.0)
        z = jnp.maximum(z @ W2[...] + b2[...], 0.0)
        return z @ W3[...] + b3[...]

    o[...] = 0.5 * (mlp(ea[...]) + mlp(eb[...]))


def _head(e2, mp):
    out = pl.pallas_call(
        _head_body,
        grid=(EM // HB,),
        in_specs=[pl.BlockSpec((HB, H), lambda i: (i, 0)),
                  pl.BlockSpec((HB, H), lambda i: (i + EM // HB, 0)),
                  _full((H, 2 * H)), _full((1, 2 * H)),
                  _full((2 * H, H)), _full((1, H)),
                  _full((H, 1)), _full((1, 1))],
        out_specs=pl.BlockSpec((HB, 1), lambda i: (i, 0)),
        out_shape=jax.ShapeDtypeStruct((EM, 1), _f32),
    )
    return out(e2, e2, mp["W1"], mp["b1"][None, :], mp["W2"], mp["b2"][None, :],
               mp["W3"], mp["b3"].reshape(1, 1))


def kernel(x, edge_index, batch, x_indicator, edge_index_mapping,
           noise_mapping_attr, t, params):
    del batch, x_indicator
    padE = EP - EU
    zpad = jnp.zeros((padE,), jnp.int32)
    npad = jnp.full((padE,), N, jnp.int32)

    src_g = jnp.concatenate([edge_index[0], zpad]).reshape(IDXR, 128)
    dst_g = jnp.concatenate([edge_index[1], npad]).reshape(IDXR, 128)
    src_u = jnp.concatenate(
        [edge_index_mapping[0], edge_index_mapping[1], zpad]).reshape(IDXR, 128)
    dst_u_raw = jnp.concatenate([edge_index_mapping[1], edge_index_mapping[0]])
    dst_u = jnp.concatenate([dst_u_raw, zpad]).reshape(IDXR, 128)
    dst_u_dummy = jnp.concatenate([dst_u_raw, npad]).reshape(IDXR, 128)

    au = jnp.concatenate([noise_mapping_attr, noise_mapping_attr,
                          jnp.zeros((padE,), _f32)]).reshape(EP, 1)
    tpad = jnp.pad(t, (0, 16 - G)).reshape(16, 1)
    zeros = jnp.zeros((512, HH), _f32)

    lp0, lp1 = params["layers"]
    te0, te1 = _time_emb(tpad, params["time"], lp0["agnn"], lp1["agnn"])

    e = _edge_embed(au, params["edge_embed"])

    aggl, aggr = _sc_gin_agg(x[:, :HH], x[:, HH:], src_g, dst_g, zeros)
    f0, xa0, xb0, xv0, xu0 = _node_dense(x, aggl, aggr, te0, lp0, first=True)
    ga = _sc_gather(xa0.reshape(N, H), src_u)
    gb = _sc_gather(xb0.reshape(N, H), dst_u)
    gv = _sc_gather(xv0.reshape(N, H), src_u)
    e, msgl, msgr = _edge_dense0(e, ga, gb, gv, lp0["agnn"])
    a2l, a2r = _sc_scatter_add(msgl, msgr, dst_u_dummy, zeros)
    x1 = _xupd(f0, xu0, a2l, a2r)

    bgl, bgr = _sc_gin_agg(x1[:, :HH], x1[:, HH:], src_g, dst_g, zeros)
    xa1, xb1 = _node_dense(x1, bgl, bgr, te1, lp1, first=False)
    ga1 = _sc_gather(xa1.reshape(N, H), src_u)
    gb1 = _sc_gather(xb1.reshape(N, H), dst_u)
    e = _edge_dense1(e, ga1, gb1, lp1["agnn"])

    return _head(e, params["map"])


# speedup vs baseline: 1.0096x; 1.0082x over previous
import functools
import math

import jax
import jax.numpy as jnp
from jax import lax
from jax.experimental import pallas as pl
from jax.experimental.pallas import tpu as pltpu
from jax.experimental.pallas import tpu_sc as plsc

H = 64
HH = H // 2
TD = 32
N = 50000
G = 10
PER = N // G
HALF = PER // 2
EG = 800000
EM = 400000
EU = 2 * EM

EP = 819200
IDXR = EP // 128
PW = EP // 32
PT = EP // 16
CHS = 128
DUMMY = N
ACC_ROWS = 50176
EB = 8192
HB = 8000

_mesh = plsc.VectorSubcoreMesh(core_axis_name="c", subcore_axis_name="s")
_f32 = jnp.float32
_cparams = pltpu.CompilerParams(use_tc_tiling_on_sc=False)


_NBUF = 4


def _sc_gather(table, idx2d):
    chw = 256
    ir = chw // 128
    nch = PW // chw

    @functools.partial(
        pl.kernel, mesh=_mesh, compiler_params=_cparams,
        out_type=jax.ShapeDtypeStruct((EP, H), jnp.bfloat16),
        scratch_types=[[pltpu.VMEM((ir, 128), jnp.int32)] * _NBUF,
                       [pltpu.VMEM((chw, H), jnp.bfloat16)] * _NBUF,
                       [pltpu.SemaphoreType.DMA] * _NBUF,
                       [pltpu.SemaphoreType.DMA] * _NBUF])
    def k(table_hbm, idx_hbm, out_hbm, ivs, rbs, gsems, ssems):
        c = lax.axis_index("c")
        s = lax.axis_index("s")
        w = c * 16 + s
        ib0 = w * (PW // 128)
        rb0 = w * PW

        def load_issue(kk, iv, rows, gsem):
            pltpu.sync_copy(idx_hbm.at[pl.ds(ib0 + kk * ir, ir)], iv)
            for j in range(ir):
                pltpu.async_copy(table_hbm.at[iv.at[j]],
                                 rows.at[pl.ds(j * 128, 128)], gsem)

        for o in range(_NBUF):
            load_issue(o, ivs[o], rbs[o], gsems[o])

        def body(g, carry):
            for o in range(_NBUF):
                kk = _NBUF * g + o
                iv, rows, gsem, ssem = ivs[o], rbs[o], gsems[o], ssems[o]
                for j in range(ir):
                    pltpu.make_async_copy(
                        table_hbm.at[iv.at[j]],
                        rows.at[pl.ds(j * 128, 128)], gsem).wait()
                out_slab = out_hbm.at[pl.ds(rb0 + kk * chw, chw)]
                pltpu.async_copy(rows, out_slab, ssem)

                @pl.when(kk + _NBUF < nch)
                def _():
                    pltpu.make_async_copy(rows, out_slab, ssem).wait()
                    load_issue(kk + _NBUF, iv, rows, gsem)
            return carry

        lax.fori_loop(0, nch // _NBUF, body, 0)
        for o in range(_NBUF):
            kk = nch - _NBUF + o
            pltpu.make_async_copy(
                rbs[o], out_hbm.at[pl.ds(rb0 + kk * chw, chw)], ssems[o]).wait()

    return k(table, idx2d)


def _acc_prologue(s, accum, zeros_hbm):
    for m in range(7):
        kz = s + 16 * m

        @pl.when(kz < ACC_ROWS // 512)
        def _():
            pltpu.sync_copy(zeros_hbm, accum.at[pl.ds(kz * 512, 512)])

    plsc.subcore_barrier()


def _acc_epilogue(s, accum, out_hbm):
    plsc.subcore_barrier()
    for m in range(4):
        kz = s + 16 * m

        @pl.when(kz < N // 1000)
        def _():
            pltpu.sync_copy(accum.at[pl.ds(kz * 1000, 1000)],
                            out_hbm.at[pl.ds(kz * 1000, 1000)])


def _sc_scatter_add(msgl, msgr, dstl, zeros):
    @functools.partial(
        pl.kernel, mesh=_mesh, compiler_params=_cparams,
        out_type=[jax.ShapeDtypeStruct((N, HH), _f32)] * 2,
        scratch_types=[pltpu.VMEM((1, 128), jnp.int32),
                       pltpu.VMEM((1, 128), jnp.int32),
                       pltpu.VMEM((CHS, HH), _f32),
                       pltpu.VMEM((CHS, HH), _f32),
                       pltpu.VMEM_SHARED((ACC_ROWS, HH), _f32),
                       pltpu.SemaphoreType.DMA, pltpu.SemaphoreType.DMA])
    def k(msgl_hbm, msgr_hbm, idx_hbm, zeros_hbm, outl_hbm, outr_hbm,
          iva, ivb, ra, rb, accum, la, lb):
        c = lax.axis_index("c")
        s = lax.axis_index("s")
        ib0 = s * (PT // 128)
        rb0 = s * PT
        nch = PT // CHS
        _acc_prologue(s, accum, zeros_hbm)

        def run(msg_hbm, out_hbm):
            def issue(kk, rows, lsem):
                pltpu.async_copy(msg_hbm.at[pl.ds(rb0 + kk * CHS, CHS)],
                                 rows, lsem)

            issue(0, ra, la)
            issue(1, rb, lb)

            def body(g, carry):
                for kk, iv, rows, lsem in ((2 * g, iva, ra, la),
                                           (2 * g + 1, ivb, rb, lb)):
                    pltpu.make_async_copy(
                        msg_hbm.at[pl.ds(rb0 + kk * CHS, CHS)], rows,
                        lsem).wait()
                    pltpu.sync_copy(idx_hbm.at[pl.ds(ib0 + kk, 1)], iv)
                    pltpu.sync_copy(rows, accum.at[iv.at[0]], add=True)

                    @pl.when(kk + 2 < nch)
                    def _():
                        issue(kk + 2, rows, lsem)
                return carry

            lax.fori_loop(0, nch // 2, body, 0)
            _acc_epilogue(s, accum, out_hbm)

        @pl.when(c == 0)
        def _():
            run(msgl_hbm, outl_hbm)

        @pl.when(c == 1)
        def _():
            run(msgr_hbm, outr_hbm)

    return k(msgl, msgr, dstl, zeros)


def _sc_gin_agg(xl, xr, src2d, dstl, zeros):
    @functools.partial(
        pl.kernel, mesh=_mesh, compiler_params=_cparams,
        out_type=[jax.ShapeDtypeStruct((N, HH), _f32)] * 2,
        scratch_types=[pltpu.VMEM((1, 128), jnp.int32),
                       pltpu.VMEM((1, 128), jnp.int32),
                       pltpu.VMEM((1, 128), jnp.int32),
                       pltpu.VMEM((1, 128), jnp.int32),
                       pltpu.VMEM((CHS, HH), _f32),
                       pltpu.VMEM((CHS, HH), _f32),
                       pltpu.VMEM_SHARED((ACC_ROWS, HH), _f32),
                       pltpu.SemaphoreType.DMA, pltpu.SemaphoreType.DMA])
    def k(xl_hbm, xr_hbm, src_hbm, idx_hbm, zeros_hbm, outl_hbm, outr_hbm,
          isa, isb, iva, ivb, ra, rb, accum, ga, gb):
        c = lax.axis_index("c")
        s = lax.axis_index("s")
        ib0 = s * (PT // 128)
        nch = PT // CHS
        _acc_prologue(s, accum, zeros_hbm)

        def run(x_hbm, out_hbm):
            def issue(kk, isv, rows, gsem):
                pltpu.sync_copy(src_hbm.at[pl.ds(ib0 + kk, 1)], isv)
                pltpu.async_copy(x_hbm.at[isv.at[0]], rows, gsem)

            issue(0, isa, ra, ga)
            issue(1, isb, rb, gb)

            def body(g, carry):
                for kk, isv, iv, rows, gsem in ((2 * g, isa, iva, ra, ga),
                                                (2 * g + 1, isb, ivb, rb, gb)):
                    pltpu.make_async_copy(x_hbm.at[isv.at[0]], rows,
                                          gsem).wait()
                    pltpu.sync_copy(idx_hbm.at[pl.ds(ib0 + kk, 1)], iv)
                    pltpu.sync_copy(rows, accum.at[iv.at[0]], add=True)

                    @pl.when(kk + 2 < nch)
                    def _():
                        issue(kk + 2, isv, rows, gsem)
                return carry

            lax.fori_loop(0, nch // 2, body, 0)
            _acc_epilogue(s, accum, out_hbm)

        @pl.when(c == 0)
        def _():
            run(xl_hbm, outl_hbm)

        @pl.when(c == 1)
        def _():
            run(xr_hbm, outr_hbm)

    return k(xl, xr, src2d, dstl, zeros)


def _full(shape):
    return pl.BlockSpec(shape, lambda *_: tuple(0 for _ in shape))


def _time_body(t_ref, w1, b1, w2, b2, T0, bT0, T1, bT1, te0, te1):
    tcol = t_ref[...]
    j = lax.broadcasted_iota(jnp.int32, (1, TD), 1).astype(_f32)
    freqs = jnp.exp(-math.log(10000.0) * j / float(TD))
    a = tcol * freqs
    emb = jnp.concatenate([jnp.cos(a), jnp.sin(a)], axis=1)
    h = jnp.maximum(emb @ w1[...] + b1[...], 0.0)
    tm = h @ w2[...] + b2[...]
    te0[...] = tm @ T0[...] + bT0[...]
    te1[...] = tm @ T1[...] + bT1[...]


def _time_emb(tpad, tp, l0, l1):
    out = pl.pallas_call(
        _time_body,
        in_specs=[_full((16, 1)), _full((H, TD)), _full((1, TD)),
                  _full((TD, TD)), _full((1, TD)),
                  _full((TD, H)), _full((1, H)), _full((TD, H)), _full((1, H))],
        out_specs=[_full((16, H)), _full((16, H))],
        out_shape=[jax.ShapeDtypeStruct((16, H), _f32)] * 2,
    )
    return out(tpad, tp["W1"], tp["b1"][None, :], tp["W2"], tp["b2"][None, :],
               l0["T"], l0["bT"][None, :], l1["T"], l1["bT"][None, :])


def _embed_body(au_ref, W, b, o_ref):
    ji = lax.broadcasted_iota(jnp.int32, (1, H), 1)
    jf = jnp.floor(ji.astype(_f32) / 2.0)
    inv_dim_t = jnp.exp(-(math.log(10000.0) * 2.0 / float(H)) * jf)
    off = jnp.where(ji % 2 == 1, math.pi / 2.0, 0.0).astype(_f32)
    pos = au_ref[...] * inv_dim_t
    o_ref[...] = jnp.sin(pos + off) @ W[...] + b[...]


def _edge_embed(au, ep):
    out = pl.pallas_call(
        _embed_body,
        grid=(EP // EB,),
        in_specs=[pl.BlockSpec((EB, 1), lambda i: (i, 0)),
                  _full((H, H)), _full((1, H))],
        out_specs=pl.BlockSpec((EB, H), lambda i: (i, 0)),
        out_shape=jax.ShapeDtypeStruct((EP, H), _f32),
    )
    return out(au, ep["W"], ep["b"][None, :])


def _node_common(x, al, ar, W1, b1, W2, b2, eps, gw, gb, gms):
    agg = jnp.concatenate([al[...][0], ar[...][0]], axis=1)
    h = (1.0 + eps[0, 0]) * x[...][0] + agg
    h = jnp.maximum(h @ W1[...] + b1[...], 0.0)
    h = h @ W2[...] + b2[...]
    mean = jnp.mean(h, axis=0, keepdims=True)
    sub = h - mean * gms[...]
    var = jnp.mean(sub * sub, axis=0, keepdims=True)
    return jnp.maximum(gw[...] * sub * jax.lax.rsqrt(var + 1e-5) + gb[...],
                       0.0)


def _node0_body(x, al, ar, te, W1, b1, W2, b2, eps, gw, gb, gms,
                A, bA, B, bB, V, bV, U, bU,
                f_o, xa_o, xb_o, xv_o, xu_o):
    f = _node_common(x, al, ar, W1, b1, W2, b2, eps, gw, gb, gms)
    f_o[...] = f[None]
    bf = jnp.bfloat16
    xa_o[...] = (f @ A[...] + bA[...] + te[...][0]).astype(bf)[None]
    xb_o[...] = (f @ B[...] + bB[...]).astype(bf)[None]
    xv_o[...] = (f @ V[...] + bV[...]).astype(bf)[None]
    xu_o[...] = (f @ U[...] + bU[...])[None]


def _node1_body(x, al, ar, te, W1, b1, W2, b2, eps, gw, gb, gms,
                A, bA, B, bB, xa_o, xb_o):
    f = _node_common(x, al, ar, W1, b1, W2, b2, eps, gw, gb, gms)
    bf = jnp.bfloat16
    xa_o[...] = (f @ A[...] + bA[...] + te[...][0]).astype(bf)[None]
    xb_o[...] = (f @ B[...] + bB[...]).astype(bf)[None]


def _node_dense(x, aggl, aggr, te, lp, first):
    g = lp["gin"]
    n = lp["gn"]
    a = lp["agnn"]
    blk = pl.BlockSpec((1, HALF, H), lambda i: (i, 0, 0))
    blkh = pl.BlockSpec((1, HALF, HH), lambda i: (i, 0, 0))
    common_specs = [blk, blkh, blkh,
                    pl.BlockSpec((1, 1, H), lambda i: (i // 2, 0, 0)),
                    _full((H, H)), _full((1, H)), _full((H, H)), _full((1, H)),
                    _full((1, 1)), _full((1, H)), _full((1, H)), _full((1, H)),
                    _full((H, H)), _full((1, H)), _full((H, H)), _full((1, H))]
    common_args = (x.reshape(N // HALF, HALF, H),
                   aggl.reshape(N // HALF, HALF, HH),
                   aggr.reshape(N // HALF, HALF, HH), te.reshape(16, 1, H),
                   g["W1"], g["b1"][None, :], g["W2"], g["b2"][None, :],
                   g["eps"].reshape(1, 1),
                   n["weight"][None, :], n["bias"][None, :],
                   n["mean_scale"][None, :],
                   a["A"], a["bA"][None, :], a["B"], a["bB"][None, :])
    if first:
        out = pl.pallas_call(
            _node0_body,
            grid=(N // HALF,),
            in_specs=common_specs + [_full((H, H)), _full((1, H)),
                                     _full((H, H)), _full((1, H))],
            out_specs=[blk] * 5,
            out_shape=[jax.ShapeDtypeStruct((N // HALF, HALF, H), _f32),
                       jax.ShapeDtypeStruct((N // HALF, HALF, H), jnp.bfloat16),
                       jax.ShapeDtypeStruct((N // HALF, HALF, H), jnp.bfloat16),
                       jax.ShapeDtypeStruct((N // HALF, HALF, H), jnp.bfloat16),
                       jax.ShapeDtypeStruct((N // HALF, HALF, H), _f32)],
        )
        return out(*common_args, a["V"], a["bV"][None, :],
                   a["U"], a["bU"][None, :])
    out = pl.pallas_call(
        _node1_body,
        grid=(N // HALF,),
        in_specs=common_specs,
        out_specs=[blk, blk],
        out_shape=[jax.ShapeDtypeStruct((N // HALF, HALF, H),
                                        jnp.bfloat16)] * 2,
    )
    return out(*common_args)


def _edge0_body(e, ga, gb, gv, C, bC, e1_o, ml_o, mr_o):
    en = (ga[...].astype(_f32) + gb[...].astype(_f32)
          + e[...] @ C[...] + bC[...])
    gate = jax.nn.sigmoid(en)
    msg = gate * gv[...].astype(_f32)
    ml_o[...] = msg[:, :HH]
    mr_o[...] = msg[:, HH:]
    e1_o[...] = e[...] + jnp.maximum(en, 0.0)


def _edge_dense0(e, ga, gb, gv, a):
    blk = pl.BlockSpec((EB, H), lambda i: (i, 0))
    blkh = pl.BlockSpec((EB, HH), lambda i: (i, 0))
    out = pl.pallas_call(
        _edge0_body,
        grid=(EP // EB,),
        in_specs=[blk, blk, blk, blk, _full((H, H)), _full((1, H))],
        out_specs=[blk, blkh, blkh],
        out_shape=[jax.ShapeDtypeStruct((EP, H), _f32),
                   jax.ShapeDtypeStruct((EP, HH), _f32),
                   jax.ShapeDtypeStruct((EP, HH), _f32)],
    )
    return out(e, ga, gb, gv, a["C"], a["bC"][None, :])


def _edge1_body(e, ga, gb, C, bC, e1_o):
    en = (ga[...].astype(_f32) + gb[...].astype(_f32)
          + e[...] @ C[...] + bC[...])
    e1_o[...] = e[...] + jnp.maximum(en, 0.0)


def _edge_dense1(e, ga, gb, a):
    blk = pl.BlockSpec((EB, H), lambda i: (i, 0))
    out = pl.pallas_call(
        _edge1_body,
        grid=(EP // EB,),
        in_specs=[blk, blk, blk, _full((H, H)), _full((1, H))],
        out_specs=blk,
        out_shape=jax.ShapeDtypeStruct((EP, H), _f32),
    )
    return out(e, ga, gb, a["C"], a["bC"][None, :])


def _xupd_body(f, xu, al, ar, o):
    agg2 = jnp.concatenate([al[...][0], ar[...][0]], axis=1)
    o[...] = f[...] + (jnp.maximum(xu[...][0] + agg2, 0.0))[None]


def _xupd(f, xu, aggl, aggr):
    blk = pl.BlockSpec((1, HALF, H), lambda i: (i, 0, 0))
    blkh = pl.BlockSpec((1, HALF, HH), lambda i: (i, 0, 0))
    out = pl.pallas_call(
        _xupd_body,
        grid=(N // HALF,),
        in_specs=[blk, blk, blkh, blkh],
        out_specs=blk,
        out_shape=jax.ShapeDtypeStruct((N // HALF, HALF, H), _f32),
    )
    return out(f, xu, aggl.reshape(N // HALF, HALF, HH),
               aggr.reshape(N // HALF, HALF, HH)).reshape(N, H)


def _head_body(ea, eb, W1, b1, W2, b2, W3, b3, o):
    def mlp(z):
        z = jnp.maximum(z @ W1[...] + b1[...], 0.0)
        z = jnp.maximum(z @ W2[...] + b2[...], 0.0)
        return z @ W3[...] + b3[...]

    o[...] = 0.5 * (mlp(ea[...]) + mlp(eb[...]))


def _head(e2, mp):
    out = pl.pallas_call(
        _head_body,
        grid=(EM // HB,),
        in_specs=[pl.BlockSpec((HB, H), lambda i: (i, 0)),
                  pl.BlockSpec((HB, H), lambda i: (i + EM // HB, 0)),
                  _full((H, 2 * H)), _full((1, 2 * H)),
                  _full((2 * H, H)), _full((1, H)),
                  _full((H, 1)), _full((1, 1))],
        out_specs=pl.BlockSpec((HB, 1), lambda i: (i, 0)),
        out_shape=jax.ShapeDtypeStruct((EM, 1), _f32),
    )
    return out(e2, e2, mp["W1"], mp["b1"][None, :], mp["W2"], mp["b2"][None, :],
               mp["W3"], mp["b3"].reshape(1, 1))


def kernel(x, edge_index, batch, x_indicator, edge_index_mapping,
           noise_mapping_attr, t, params):
    del batch, x_indicator
    padE = EP - EU
    zpad = jnp.zeros((padE,), jnp.int32)
    npad = jnp.full((padE,), N, jnp.int32)

    src_g = jnp.concatenate([edge_index[0], zpad]).reshape(IDXR, 128)
    dst_g = jnp.concatenate([edge_index[1], npad]).reshape(IDXR, 128)
    src_u = jnp.concatenate(
        [edge_index_mapping[0], edge_index_mapping[1], zpad]).reshape(IDXR, 128)
    dst_u_raw = jnp.concatenate([edge_index_mapping[1], edge_index_mapping[0]])
    dst_u = jnp.concatenate([dst_u_raw, zpad]).reshape(IDXR, 128)
    dst_u_dummy = jnp.concatenate([dst_u_raw, npad]).reshape(IDXR, 128)

    au = jnp.concatenate([noise_mapping_attr, noise_mapping_attr,
                          jnp.zeros((padE,), _f32)]).reshape(EP, 1)
    tpad = jnp.pad(t, (0, 16 - G)).reshape(16, 1)
    zeros = jnp.zeros((512, HH), _f32)

    lp0, lp1 = params["layers"]
    te0, te1 = _time_emb(tpad, params["time"], lp0["agnn"], lp1["agnn"])

    e = _edge_embed(au, params["edge_embed"])

    aggl, aggr = _sc_gin_agg(x[:, :HH], x[:, HH:], src_g, dst_g, zeros)
    f0, xa0, xb0, xv0, xu0 = _node_dense(x, aggl, aggr, te0, lp0, first=True)
    ga = _sc_gather(xa0.reshape(N, H), src_u)
    gb = _sc_gather(xb0.reshape(N, H), dst_u)
    gv = _sc_gather(xv0.reshape(N, H), src_u)
    e, msgl, msgr = _edge_dense0(e, ga, gb, gv, lp0["agnn"])
    a2l, a2r = _sc_scatter_add(msgl, msgr, dst_u_dummy, zeros)
    x1 = _xupd(f0, xu0, a2l, a2r)

    bgl, bgr = _sc_gin_agg(x1[:, :HH], x1[:, HH:], src_g, dst_g, zeros)
    xa1, xb1 = _node_dense(x1, bgl, bgr, te1, lp1, first=False)
    ga1 = _sc_gather(xa1.reshape(N, H), src_u)
    gb1 = _sc_gather(xb1.reshape(N, H), dst_u)
    e = _edge_dense1(e, ga1, gb1, lp1["agnn"])

    return _head(e, params["map"])
